# baseline (speedup 1.0000x reference)
import jax
import jax.numpy as jnp
from jax import lax
from jax.experimental import pallas as pl
from jax.experimental.pallas import tpu as pltpu
from jax.experimental.pallas import tpu_sc as plsc

DIM = 384
NPAIR = 100 * 50
NC, NS, L = 2, 16, 16
NW = NC * NS
K = 160
NCHUNK = 2
CK = K // NCHUNK
NACC = DIM // L


def _sc_body(table_hbm, idx_hbm, w_hbm, out_hbm,
             idx_v, w_v, rows_v, acc_v, sem0, sem1, sem2):
    wid = lax.axis_index("s") * NC + lax.axis_index("c")
    vstart = wid * K
    base = jnp.minimum(vstart, NPAIR - K)

    ci0 = pltpu.async_copy(idx_hbm.at[pl.ds(base, CK)], idx_v.at[0], sem0)
    ci1 = pltpu.async_copy(idx_hbm.at[pl.ds(base + CK, CK)], idx_v.at[1],
                           sem1)
    cw0 = pltpu.async_copy(w_hbm.at[pl.ds(base, CK)], w_v.at[0], sem2)
    cw1 = pltpu.async_copy(w_hbm.at[pl.ds(base + CK, CK)], w_v.at[1], sem2)
    H = CK // 2
    ci0.wait()
    cp0a = pltpu.async_copy(table_hbm.at[idx_v.at[0, pl.ds(0, H)]],
                            rows_v.at[0, pl.ds(0, H)], sem0)
    cp0b = pltpu.async_copy(table_hbm.at[idx_v.at[0, pl.ds(H, H)]],
                            rows_v.at[0, pl.ds(H, H)], sem0)
    ci1.wait()
    cp1a = pltpu.async_copy(table_hbm.at[idx_v.at[1, pl.ds(0, H)]],
                            rows_v.at[1, pl.ds(0, H)], sem1)
    cp1b = pltpu.async_copy(table_hbm.at[idx_v.at[1, pl.ds(H, H)]],
                            rows_v.at[1, pl.ds(H, H)], sem1)

    zero = jnp.zeros((L,), jnp.float32)
    for c in range(NACC):
        acc_v[pl.ds(c * L, L)] = zero
    cw0.wait()
    cw1.wait()
    lane = lax.iota(jnp.int32, L)
    for j in range(NCHUNK):
        for g in range(CK // L):
            pair = (base + j * CK + g * L) + lane
            w16 = w_v[j, pl.ds(g * L, L)]
            w_v[j, pl.ds(g * L, L)] = jnp.where(pair >= vstart, w16, 0.0)

    def group_body(j):
        def body(g, carry):
            base_r = g * L
            w16 = w_v[j, pl.ds(base_r, L)]
            third = NACC // 3
            for h in range(3):
                acc = [None] * third
                for r in range(L):
                    wv = lax.gather(
                        w16, jnp.full((L, 1), r, jnp.int32),
                        lax.GatherDimensionNumbers(offset_dims=(),
                                                   collapsed_slice_dims=(0,),
                                                   start_index_map=(0,)),
                        slice_sizes=(1,),
                        mode=lax.GatherScatterMode.PROMISE_IN_BOUNDS)
                    for ci in range(third):
                        c = h * third + ci
                        t = wv * rows_v[j, base_r + r, pl.ds(c * L, L)]
                        acc[ci] = t if acc[ci] is None else acc[ci] + t
                for ci in range(third):
                    c = h * third + ci
                    plsc.addupdate(acc_v.at[pl.ds(c * L, L)], acc[ci])
            return carry
        return body

    cp0a.wait()
    cp0b.wait()
    lax.fori_loop(0, CK // L, group_body(0), 0)
    cp1a.wait()
    cp1b.wait()
    lax.fori_loop(0, CK // L, group_body(1), 0)

    pltpu.sync_copy(acc_v, out_hbm.at[wid])


_sc_partials = pl.kernel(
    _sc_body,
    out_type=jax.ShapeDtypeStruct((NW, DIM), jnp.float32),
    mesh=plsc.VectorSubcoreMesh(core_axis_name="c", subcore_axis_name="s",
                                num_cores=NC, num_subcores=NS),
    scratch_types=[
        pltpu.VMEM((NCHUNK, CK), jnp.int32),
        pltpu.VMEM((NCHUNK, CK), jnp.float32),
        pltpu.VMEM((NCHUNK, CK, DIM), jnp.float32),
        pltpu.VMEM((DIM,), jnp.float32),
        pltpu.SemaphoreType.DMA,
        pltpu.SemaphoreType.DMA,
        pltpu.SemaphoreType.DMA,
    ],
)


def _finish_body(parts_ref, out_ref):
    s = jnp.sum(parts_ref[...], axis=0, keepdims=True)
    ss = jnp.sum(s * s)
    out_ref[...] = s * lax.rsqrt(ss)


_finish = pl.pallas_call(
    _finish_body,
    out_shape=jax.ShapeDtypeStruct((1, DIM), jnp.float32),
)


def kernel(subject_table, subject_idx, subtopic_weights, topic_weights):
    idx_flat = subject_idx.reshape(-1).astype(jnp.int32)
    w_flat = (topic_weights[:, None] * subtopic_weights).reshape(-1)
    parts = _sc_partials(subject_table, idx_flat, w_flat)
    return _finish(parts).reshape(DIM)


# speedup vs baseline: 1.0074x; 1.0074x over previous
import jax
import jax.numpy as jnp
from jax import lax
from jax.experimental import pallas as pl
from jax.experimental.pallas import tpu as pltpu
from jax.experimental.pallas import tpu_sc as plsc

DIM = 384
NPAIR = 100 * 50
NC, NS, L = 2, 16, 16
NW = NC * NS
K = 160
NCHUNK = 2
CK = K // NCHUNK
NACC = DIM // L


def _sc_body(table_hbm, idx_hbm, w_hbm, out_hbm,
             idx_v, w_v, rows_v, acc_v, sem0, sem1, sem2):
    wid = lax.axis_index("s") * NC + lax.axis_index("c")
    vstart = wid * K
    base = jnp.minimum(vstart, NPAIR - K)

    ci0 = pltpu.async_copy(idx_hbm.at[pl.ds(base, CK)], idx_v.at[0], sem0)
    ci1 = pltpu.async_copy(idx_hbm.at[pl.ds(base + CK, CK)], idx_v.at[1],
                           sem1)
    cw0 = pltpu.async_copy(w_hbm.at[pl.ds(base, CK)], w_v.at[0], sem2)
    cw1 = pltpu.async_copy(w_hbm.at[pl.ds(base + CK, CK)], w_v.at[1], sem2)
    ci0.wait()
    cp0 = pltpu.async_copy(table_hbm.at[idx_v.at[0]], rows_v.at[0], sem0)
    ci1.wait()
    cp1 = pltpu.async_copy(table_hbm.at[idx_v.at[1]], rows_v.at[1], sem1)

    zero = jnp.zeros((L,), jnp.float32)
    for c in range(NACC):
        acc_v[pl.ds(c * L, L)] = zero
    cw0.wait()
    cw1.wait()
    lane = lax.iota(jnp.int32, L)
    for j in range(NCHUNK):
        for g in range(CK // L):
            pair = (base + j * CK + g * L) + lane
            w16 = w_v[j, pl.ds(g * L, L)]
            w_v[j, pl.ds(g * L, L)] = jnp.where(pair >= vstart, w16, 0.0)

    def group_body(j):
        def body(g, carry):
            base_r = g * L
            w16 = w_v[j, pl.ds(base_r, L)]
            third = NACC // 3
            for h in range(3):
                acc = [None] * third
                for r in range(L):
                    wv = lax.gather(
                        w16, jnp.full((L, 1), r, jnp.int32),
                        lax.GatherDimensionNumbers(offset_dims=(),
                                                   collapsed_slice_dims=(0,),
                                                   start_index_map=(0,)),
                        slice_sizes=(1,),
                        mode=lax.GatherScatterMode.PROMISE_IN_BOUNDS)
                    for ci in range(third):
                        c = h * third + ci
                        t = wv * rows_v[j, base_r + r, pl.ds(c * L, L)]
                        acc[ci] = t if acc[ci] is None else acc[ci] + t
                for ci in range(third):
                    c = h * third + ci
                    plsc.addupdate(acc_v.at[pl.ds(c * L, L)], acc[ci])
            return carry
        return body

    cp0.wait()
    lax.fori_loop(0, CK // L, group_body(0), 0)
    cp1.wait()
    lax.fori_loop(0, CK // L, group_body(1), 0)

    pltpu.sync_copy(acc_v, out_hbm.at[wid])


_sc_partials = pl.kernel(
    _sc_body,
    out_type=jax.ShapeDtypeStruct((NW, DIM), jnp.float32),
    mesh=plsc.VectorSubcoreMesh(core_axis_name="c", subcore_axis_name="s",
                                num_cores=NC, num_subcores=NS),
    scratch_types=[
        pltpu.VMEM((NCHUNK, CK), jnp.int32),
        pltpu.VMEM((NCHUNK, CK), jnp.float32),
        pltpu.VMEM((NCHUNK, CK, DIM), jnp.float32),
        pltpu.VMEM((DIM,), jnp.float32),
        pltpu.SemaphoreType.DMA,
        pltpu.SemaphoreType.DMA,
        pltpu.SemaphoreType.DMA,
    ],
)


def _finish_body(parts_ref, out_ref):
    s = jnp.sum(parts_ref[...], axis=0, keepdims=True)
    ss = jnp.sum(s * s)
    out_ref[...] = s * lax.rsqrt(ss)


_finish = pl.pallas_call(
    _finish_body,
    out_shape=jax.ShapeDtypeStruct((1, DIM), jnp.float32),
)


def kernel(subject_table, subject_idx, subtopic_weights, topic_weights):
    idx_flat = subject_idx.reshape(-1).astype(jnp.int32)
    w_flat = (topic_weights[:, None] * subtopic_weights).reshape(-1)
    parts = _sc_partials(subject_table, idx_flat, w_flat)
    return _finish(parts).reshape(DIM)


# speedup vs baseline: 1.0436x; 1.0359x over previous
import jax
import jax.numpy as jnp
from jax import lax
from jax.experimental import pallas as pl
from jax.experimental.pallas import tpu as pltpu
from jax.experimental.pallas import tpu_sc as plsc

DIM = 384
NPAIR = 100 * 50
NC, NS, L = 2, 16, 16
NW = NC * NS
K = 160
NCHUNK = 2
CK = K // NCHUNK
NACC = DIM // L
GR = 8
NG = K // GR


def _bcast(w16, idx):
    return lax.gather(
        w16, jnp.full((L,), idx, jnp.int32)[:, None],
        lax.GatherDimensionNumbers(offset_dims=(),
                                   collapsed_slice_dims=(0,),
                                   start_index_map=(0,)),
        slice_sizes=(1,),
        mode=lax.GatherScatterMode.PROMISE_IN_BOUNDS)


def _sc_body(table_hbm, idx_hbm, w_hbm, out_hbm,
             idx_v, w_v, rows_v, acc_v, sem0, sem1, sem2):
    wid = lax.axis_index("s") * NC + lax.axis_index("c")
    vstart = wid * K
    base = jnp.minimum(vstart, NPAIR - K)

    ci0 = pltpu.async_copy(idx_hbm.at[pl.ds(base, CK)], idx_v.at[0], sem0)
    ci1 = pltpu.async_copy(idx_hbm.at[pl.ds(base + CK, CK)], idx_v.at[1],
                           sem1)
    cw = pltpu.async_copy(w_hbm.at[pl.ds(base, K)], w_v, sem2)
    ci0.wait()
    cp0 = pltpu.async_copy(table_hbm.at[idx_v.at[0]], rows_v.at[pl.ds(0, CK)],
                           sem0)
    ci1.wait()
    cp1 = pltpu.async_copy(table_hbm.at[idx_v.at[1]],
                           rows_v.at[pl.ds(CK, CK)], sem1)

    zero = jnp.zeros((L,), jnp.float32)
    for c in range(NACC):
        acc_v[pl.ds(c * L, L)] = zero

    lane = lax.iota(jnp.int32, L)
    cw.wait()
    cp0.wait()
    cp1.wait()

    def body(g, carry):
        off16 = (g // 2) * L
        rbase = (g % 2) * GR
        rowb = g * GR
        pair = (base + off16) + lane
        w16 = jnp.where(pair >= vstart, w_v[pl.ds(off16, L)], 0.0)
        third = NACC // 3
        for h in range(3):
            acc = [None] * third
            for r in range(GR):
                wv = _bcast(w16, rbase + r)
                for ci in range(third):
                    c = h * third + ci
                    t = wv * rows_v[rowb + r, pl.ds(c * L, L)]
                    acc[ci] = t if acc[ci] is None else acc[ci] + t
            for ci in range(third):
                c = h * third + ci
                plsc.addupdate(acc_v.at[pl.ds(c * L, L)], acc[ci])
        return carry

    lax.fori_loop(0, NG, body, 0)
    pltpu.sync_copy(acc_v, out_hbm.at[wid])


_sc_partials = pl.kernel(
    _sc_body,
    out_type=jax.ShapeDtypeStruct((NW, DIM), jnp.float32),
    mesh=plsc.VectorSubcoreMesh(core_axis_name="c", subcore_axis_name="s",
                                num_cores=NC, num_subcores=NS),
    scratch_types=[
        pltpu.VMEM((NCHUNK, CK), jnp.int32),
        pltpu.VMEM((K,), jnp.float32),
        pltpu.VMEM((K, DIM), jnp.float32),
        pltpu.VMEM((DIM,), jnp.float32),
        pltpu.SemaphoreType.DMA,
        pltpu.SemaphoreType.DMA,
        pltpu.SemaphoreType.DMA,
    ],
)


def _finish_body(parts_ref, out_ref):
    s = jnp.sum(parts_ref[...], axis=0, keepdims=True)
    ss = jnp.sum(s * s)
    out_ref[...] = s * lax.rsqrt(ss)


_finish = pl.pallas_call(
    _finish_body,
    out_shape=jax.ShapeDtypeStruct((1, DIM), jnp.float32),
)


def kernel(subject_table, subject_idx, subtopic_weights, topic_weights):
    idx_flat = subject_idx.reshape(-1).astype(jnp.int32)
    w_flat = (topic_weights[:, None] * subtopic_weights).reshape(-1)
    parts = _sc_partials(subject_table, idx_flat, w_flat)
    return _finish(parts).reshape(DIM)
